# Initial kernel scaffold; baseline (speedup 1.0000x reference)
import jax
import jax.numpy as jnp
from jax.experimental import pallas as pl


def kernel(input, pos_con, neg_con, input_table, context_table):
    raise NotImplementedError("write your pallas kernel here")


# speedup vs baseline: 1.4477x; 1.4477x over previous
import functools

import jax
import jax.numpy as jnp
from jax import lax
from jax.experimental import pallas as pl
from jax.experimental.pallas import tpu as pltpu
from jax.experimental.pallas import tpu_sc as plsc

B = 4096
K = 5
D = 128
NC = 2
NS = 16
NW = NC * NS
CPW = B // NW
SPW = CPW * (K + 1)
NCHUNK = D // 16


def _sc_body(in_idx_hbm, pos_idx_hbm, neg_idx_hbm, in_tab_hbm, ctx_tab_hbm,
             out_hbm, in_idx_v, pos_idx_v, neg_idx_v, in_rows_v, pos_rows_v,
             neg_rows_v, scores_v, sem):
    wid = lax.axis_index("s") * NC + lax.axis_index("c")
    base = wid * CPW

    pltpu.sync_copy(in_idx_hbm.at[pl.ds(base, CPW)], in_idx_v)
    pltpu.sync_copy(pos_idx_hbm.at[pl.ds(base, CPW)], pos_idx_v)
    for k in range(K):
        pltpu.sync_copy(neg_idx_hbm.at[pl.ds(k * B + base, CPW)],
                        neg_idx_v.at[k])

    copies = [
        pltpu.async_copy(in_tab_hbm.at[in_idx_v], in_rows_v, sem),
        pltpu.async_copy(ctx_tab_hbm.at[pos_idx_v], pos_rows_v, sem),
    ]
    for k in range(K):
        copies.append(
            pltpu.async_copy(ctx_tab_hbm.at[neg_idx_v.at[k]],
                             neg_rows_v.at[k], sem))
    for c in copies:
        c.wait()

    lanes = lax.iota(jnp.int32, 16)

    def group(g, carry):
        vecs = [jnp.zeros((16,), jnp.float32) for _ in range(K + 1)]
        for j in range(16):
            i = g * 16 + j
            a = [in_rows_v[i, pl.ds(d * 16, 16)] for d in range(NCHUNK)]
            acc = a[0] * pos_rows_v[i, pl.ds(0, 16)]
            for d in range(1, NCHUNK):
                acc = acc + a[d] * pos_rows_v[i, pl.ds(d * 16, 16)]
            vecs[0] = jnp.where(lanes == j, jnp.sum(acc), vecs[0])
            for k in range(K):
                acc = a[0] * neg_rows_v[k, i, pl.ds(0, 16)]
                for d in range(1, NCHUNK):
                    acc = acc + a[d] * neg_rows_v[k, i, pl.ds(d * 16, 16)]
                vecs[k + 1] = jnp.where(lanes == j, -jnp.sum(acc), vecs[k + 1])
        for s in range(K + 1):
            scores_v[pl.ds(s * CPW + g * 16, 16)] = vecs[s]
        return carry

    lax.fori_loop(0, CPW // 16, group, 0)
    pltpu.sync_copy(scores_v, out_hbm.at[pl.ds(wid * SPW, SPW)])


_sc_scores = functools.partial(
    pl.kernel,
    mesh=plsc.VectorSubcoreMesh(core_axis_name="c", subcore_axis_name="s"),
    compiler_params=pltpu.CompilerParams(needs_layout_passes=False),
    out_type=jax.ShapeDtypeStruct((B * (K + 1),), jnp.float32),
    scratch_types=[
        pltpu.VMEM((CPW,), jnp.int32),
        pltpu.VMEM((CPW,), jnp.int32),
        pltpu.VMEM((K, CPW), jnp.int32),
        pltpu.VMEM((CPW, D), jnp.float32),
        pltpu.VMEM((CPW, D), jnp.float32),
        pltpu.VMEM((K, CPW, D), jnp.float32),
        pltpu.VMEM((SPW,), jnp.float32),
        pltpu.SemaphoreType.DMA,
    ],
)(_sc_body)


def _tc_loss_body(x_ref, o_ref):
    x = x_ref[...]
    z = -x
    sp = jnp.maximum(z, 0.0) + jnp.log(1.0 + jnp.exp(-jnp.abs(z)))
    o_ref[0, 0] = jnp.sum(sp) / (B * (K + 1))


_tc_loss = pl.pallas_call(
    _tc_loss_body,
    out_shape=jax.ShapeDtypeStruct((1, 1), jnp.float32),
    out_specs=pl.BlockSpec(memory_space=pltpu.SMEM),
)


@jax.jit
def kernel(input, pos_con, neg_con, input_table, context_table):
    in_idx = input.reshape(-1).astype(jnp.int32)
    pos_idx = pos_con.reshape(-1).astype(jnp.int32)
    neg_idx = neg_con.reshape(-1).astype(jnp.int32)
    scores = _sc_scores(in_idx, pos_idx, neg_idx, input_table, context_table)
    loss = _tc_loss(scores.reshape(B * (K + 1) // D, D))
    return loss.reshape(())


# speedup vs baseline: 1.6753x; 1.1572x over previous
import functools

import jax
import jax.numpy as jnp
from jax import lax
from jax.experimental import pallas as pl
from jax.experimental.pallas import tpu as pltpu
from jax.experimental.pallas import tpu_sc as plsc

B = 4096
K = 5
D = 128
NC = 2
NS = 16
NW = NC * NS
CPW = B // NW
SPW = CPW * (K + 1)
NCHUNK = D // 16


def _sc_body(in_idx_hbm, pos_idx_hbm, neg_idx_hbm, in_tab_hbm, ctx_tab_hbm,
             out_hbm, in_idx_v, pos_idx_v, neg_idx_v, in_rows_v, pos_rows_v,
             neg_rows_v, scores_v, sem):
    wid = lax.axis_index("s") * NC + lax.axis_index("c")
    base = wid * CPW

    pltpu.sync_copy(in_idx_hbm.at[pl.ds(base, CPW)], in_idx_v)
    pltpu.sync_copy(pos_idx_hbm.at[pl.ds(base, CPW)], pos_idx_v)
    for k in range(K):
        pltpu.sync_copy(neg_idx_hbm.at[pl.ds(k * B + base, CPW)],
                        neg_idx_v.at[k])

    copies = [
        pltpu.async_copy(in_tab_hbm.at[in_idx_v], in_rows_v, sem),
        pltpu.async_copy(ctx_tab_hbm.at[pos_idx_v], pos_rows_v, sem),
    ]
    for k in range(K):
        copies.append(
            pltpu.async_copy(ctx_tab_hbm.at[neg_idx_v.at[k]],
                             neg_rows_v.at[k], sem))
    for c in copies:
        c.wait()

    lanes = lax.iota(jnp.int32, 16)
    zero = jnp.zeros((16,), jnp.float32)

    def group(g, carry):
        def lane(j, vecs):
            i = g * 16 + j
            a = [in_rows_v[i, pl.ds(d * 16, 16)] for d in range(NCHUNK)]
            acc = a[0] * pos_rows_v[i, pl.ds(0, 16)]
            for d in range(1, NCHUNK):
                acc = acc + a[d] * pos_rows_v[i, pl.ds(d * 16, 16)]
            out = [jnp.where(lanes == j, jnp.sum(acc), vecs[0])]
            for k in range(K):
                acc = a[0] * neg_rows_v[k, i, pl.ds(0, 16)]
                for d in range(1, NCHUNK):
                    acc = acc + a[d] * neg_rows_v[k, i, pl.ds(d * 16, 16)]
                out.append(jnp.where(lanes == j, -jnp.sum(acc), vecs[k + 1]))
            return tuple(out)

        vecs = lax.fori_loop(0, 16, lane, (zero,) * (K + 1))
        for s in range(K + 1):
            scores_v[pl.ds(s * CPW + g * 16, 16)] = vecs[s]
        return carry

    lax.fori_loop(0, CPW // 16, group, 0)
    pltpu.sync_copy(scores_v, out_hbm.at[pl.ds(wid * SPW, SPW)])


_sc_scores = functools.partial(
    pl.kernel,
    mesh=plsc.VectorSubcoreMesh(core_axis_name="c", subcore_axis_name="s"),
    compiler_params=pltpu.CompilerParams(needs_layout_passes=False),
    out_type=jax.ShapeDtypeStruct((B * (K + 1),), jnp.float32),
    scratch_types=[
        pltpu.VMEM((CPW,), jnp.int32),
        pltpu.VMEM((CPW,), jnp.int32),
        pltpu.VMEM((K, CPW), jnp.int32),
        pltpu.VMEM((CPW, D), jnp.float32),
        pltpu.VMEM((CPW, D), jnp.float32),
        pltpu.VMEM((K, CPW, D), jnp.float32),
        pltpu.VMEM((SPW,), jnp.float32),
        pltpu.SemaphoreType.DMA,
    ],
)(_sc_body)


def _tc_loss_body(x_ref, o_ref):
    x = x_ref[...]
    z = -x
    sp = jnp.maximum(z, 0.0) + jnp.log(1.0 + jnp.exp(-jnp.abs(z)))
    o_ref[0, 0] = jnp.sum(sp) / (B * (K + 1))


_tc_loss = pl.pallas_call(
    _tc_loss_body,
    out_shape=jax.ShapeDtypeStruct((1, 1), jnp.float32),
    out_specs=pl.BlockSpec(memory_space=pltpu.SMEM),
)


@jax.jit
def kernel(input, pos_con, neg_con, input_table, context_table):
    in_idx = input.reshape(-1).astype(jnp.int32)
    pos_idx = pos_con.reshape(-1).astype(jnp.int32)
    neg_idx = neg_con.reshape(-1).astype(jnp.int32)
    scores = _sc_scores(in_idx, pos_idx, neg_idx, input_table, context_table)
    loss = _tc_loss(scores.reshape(B * (K + 1) // D, D))
    return loss.reshape(())
